# baseline (speedup 1.0000x reference)
import functools

import jax
import jax.numpy as jnp
from jax import lax
from jax.experimental import pallas as pl
from jax.experimental.pallas import tpu as pltpu
from jax.experimental.pallas import tpu_sc as plsc

VOCAB = 1000000
EMBED_DIM = 32
BATCH = 16384
_PACK = 128 // EMBED_DIM

_info = plsc.get_sparse_core_info()
_NC, _NS = _info.num_cores, _info.num_subcores
_NW = _NC * _NS
_B_PER_W = BATCH // _NW
_CHUNK = 128
_N_CHUNKS = _B_PER_W // _CHUNK
_L = 16

_mesh = plsc.VectorSubcoreMesh(core_axis_name="c", subcore_axis_name="s")


@functools.partial(
    pl.kernel,
    mesh=_mesh,
    out_type=jax.ShapeDtypeStruct((BATCH, EMBED_DIM), jnp.float32),
    compiler_params=pltpu.CompilerParams(needs_layout_passes=False),
    scratch_types=[
        pltpu.VMEM((_N_CHUNKS, _CHUNK), jnp.int32),
        pltpu.VMEM((_B_PER_W,), jnp.int32),
        pltpu.VMEM((2, _CHUNK, 128), jnp.float32),
        pltpu.VMEM((_B_PER_W, EMBED_DIM), jnp.float32),
        pltpu.SemaphoreType.DMA,
    ],
)
def _gather_kernel(table_hbm, idx_hbm, out_hbm, idxg_v, sub_v, lines_v,
                   rows_v, sem):
    wid = lax.axis_index("s") * _NC + lax.axis_index("c")
    pltpu.sync_copy(idx_hbm.at[wid], idxg_v)
    for k in range(_B_PER_W // _L):
        j, o = divmod(k * _L, _CHUNK)
        v = idxg_v[j, pl.ds(o, _L)]
        sub_v[pl.ds(k * _L, _L)] = (v & (_PACK - 1)) * EMBED_DIM
        idxg_v[j, pl.ds(o, _L)] = v >> 2

    iota = lax.iota(jnp.int32, _L)

    def fire(c):
        return pltpu.async_copy(
            table_hbm.at[idxg_v.at[c]], lines_v.at[c % 2], sem
        )

    def extract(c):
        buf = lines_v.at[c % 2]

        def body(i, carry):
            splat_i = jnp.full((_L,), i, jnp.int32)
            off = plsc.load_gather(sub_v, [c * _CHUNK + splat_i])
            for h in range(EMBED_DIM // _L):
                vals = plsc.load_gather(buf, [splat_i, off + (h * _L + iota)])
                rows_v[c * _CHUNK + i, pl.ds(h * _L, _L)] = vals
            return carry

        lax.fori_loop(0, _CHUNK, body, 0)

    copies = [fire(0), fire(1)]
    for c in range(_N_CHUNKS):
        copies[c].wait()
        extract(c)
        if c + 2 < _N_CHUNKS:
            copies.append(fire(c + 2))

    pltpu.sync_copy(rows_v, out_hbm.at[pl.ds(wid * _B_PER_W, _B_PER_W)])


def kernel(indices, table):
    idx = indices.astype(jnp.int32).reshape(_NW, _N_CHUNKS, _CHUNK)
    lines = table.reshape(VOCAB // _PACK, 128)
    return _gather_kernel(lines, idx)


# speedup vs baseline: 1.6242x; 1.6242x over previous
import functools

import jax
import jax.numpy as jnp
from jax import lax
from jax.experimental import pallas as pl
from jax.experimental.pallas import tpu as pltpu
from jax.experimental.pallas import tpu_sc as plsc

VOCAB = 1000000
EMBED_DIM = 32
BATCH = 16384

_info = plsc.get_sparse_core_info()
_NC, _NS = _info.num_cores, _info.num_subcores
_NW = _NC * _NS
_B_PER_W = BATCH // _NW
_BLK = 16
_N_BLK = _B_PER_W // _BLK

_mesh = plsc.VectorSubcoreMesh(core_axis_name="c", subcore_axis_name="s")


@functools.partial(
    pl.kernel,
    mesh=_mesh,
    out_type=jax.ShapeDtypeStruct((BATCH, EMBED_DIM), jnp.float32),
    compiler_params=pltpu.CompilerParams(needs_layout_passes=False),
    scratch_types=[
        pltpu.VMEM((_B_PER_W,), jnp.int32),
        pltpu.VMEM((_B_PER_W, EMBED_DIM), jnp.float32),
        pltpu.SemaphoreType.DMA,
    ],
)
def _gather_kernel(table_hbm, idx_hbm, out_hbm, idx_v, rows_v, sem):
    wid = lax.axis_index("s") * _NC + lax.axis_index("c")
    pltpu.sync_copy(idx_hbm.at[wid], idx_v)

    def drain_block(b):
        pltpu.make_async_copy(
            table_hbm.at[pl.ds(0, _BLK)], rows_v.at[pl.ds(b * _BLK, _BLK)], sem
        ).wait()

    def body(j, carry):
        base = j * _BLK
        v = idx_v[pl.ds(base, _BLK)]
        for t in range(_BLK):
            r = v[t]
            pltpu.async_copy(
                table_hbm.at[pl.ds(r, 1)],
                rows_v.at[pl.ds(base + t, 1)],
                sem,
            )

        @pl.when(j > 0)
        def _():
            drain_block(j - 1)

        return carry

    lax.fori_loop(0, _N_BLK, body, 0)
    drain_block(_N_BLK - 1)

    pltpu.sync_copy(rows_v, out_hbm.at[pl.ds(wid * _B_PER_W, _B_PER_W)])


def kernel(indices, table):
    idx = indices.astype(jnp.int32).reshape(_NW, _B_PER_W)
    return _gather_kernel(table, idx)


# speedup vs baseline: 1.6736x; 1.0304x over previous
import functools

import jax
import jax.numpy as jnp
from jax import lax
from jax.experimental import pallas as pl
from jax.experimental.pallas import tpu as pltpu
from jax.experimental.pallas import tpu_sc as plsc

VOCAB = 1000000
EMBED_DIM = 32
BATCH = 16384

_info = plsc.get_sparse_core_info()
_NC, _NS = _info.num_cores, _info.num_subcores
_NW = _NC * _NS
_B_PER_W = BATCH // _NW
_BLK = 16
_N_BLK = _B_PER_W // _BLK

_mesh = plsc.VectorSubcoreMesh(core_axis_name="c", subcore_axis_name="s")


@functools.partial(
    pl.kernel,
    mesh=_mesh,
    out_type=jax.ShapeDtypeStruct((BATCH, EMBED_DIM), jnp.float32),
    compiler_params=pltpu.CompilerParams(needs_layout_passes=False),
    scratch_types=[
        pltpu.VMEM((_B_PER_W,), jnp.int32),
        pltpu.VMEM((_B_PER_W, EMBED_DIM), jnp.float32),
        pltpu.SemaphoreType.DMA,
    ],
)
def _gather_kernel(table_hbm, idx_hbm, out_hbm, idx_v, rows_v, sem):
    wid = lax.axis_index("s") * _NC + lax.axis_index("c")
    pltpu.sync_copy(idx_hbm.at[wid], idx_v)

    def body(j, carry):
        base = j * _BLK
        v = idx_v[pl.ds(base, _BLK)]
        for t in range(_BLK):
            r = v[t]
            pltpu.async_copy(
                table_hbm.at[pl.ds(r, 1)],
                rows_v.at[pl.ds(base + t, 1)],
                sem,
            )
        return carry

    lax.fori_loop(0, _N_BLK, body, 0)
    pltpu.make_async_copy(table_hbm.at[pl.ds(0, _B_PER_W)], rows_v, sem).wait()

    pltpu.sync_copy(rows_v, out_hbm.at[pl.ds(wid * _B_PER_W, _B_PER_W)])


def kernel(indices, table):
    idx = indices.astype(jnp.int32).reshape(_NW, _B_PER_W)
    return _gather_kernel(table, idx)
